# baseline (speedup 1.0000x reference)
import functools

import jax
import jax.numpy as jnp
from jax import lax
from jax.experimental import pallas as pl
from jax.experimental.pallas import tpu as pltpu
from jax.experimental.pallas import tpu_sc as plsc

NC = 2
NS = 16
NW = NC * NS
CHUNK = 128


def _elu(v):
    return jnp.where(v > 0, v, jnp.exp(jnp.minimum(v, 0.0)) - 1.0)


def _make_deg_kernel(n_pad, d, n_chunks_total):
    per_tile = n_chunks_total // NW
    rps = n_pad // NS
    mesh = plsc.VectorSubcoreMesh(core_axis_name="c", subcore_axis_name="s",
                                  num_cores=NC, num_subcores=NS)

    @functools.partial(
        pl.kernel,
        out_type=jax.ShapeDtypeStruct((NC, n_pad, d), jnp.float32),
        mesh=mesh,
        scratch_types=[
            pltpu.VMEM((per_tile, CHUNK), jnp.int32),
            pltpu.VMEM((CHUNK, d), jnp.float32),
            pltpu.VMEM_SHARED((n_pad, d), jnp.float32),
            pltpu.SemaphoreType.DMA,
        ],
    )
    def deg_kernel(col2d, ones_hbm, zeros_hbm, out_hbm, cidx, ones_v, acc, sem):
        c = lax.axis_index("c")
        s = lax.axis_index("s")
        tile = c * NS + s
        pltpu.sync_copy(zeros_hbm.at[pl.ds(s * rps, rps)], acc.at[pl.ds(s * rps, rps)])
        pltpu.sync_copy(col2d.at[pl.ds(tile * per_tile, per_tile)], cidx)
        pltpu.sync_copy(ones_hbm, ones_v)
        plsc.subcore_barrier()

        def body(j, carry):
            pltpu.async_copy(ones_v, acc.at[cidx.at[j]], sem, add=True)
            return carry

        lax.fori_loop(0, per_tile, body, 0)

        def drain(j, carry):
            pltpu.make_async_copy(ones_v, acc.at[cidx.at[j]], sem).wait()
            return carry

        lax.fori_loop(0, per_tile, drain, 0)
        plsc.subcore_barrier()
        pltpu.sync_copy(acc.at[pl.ds(s * rps, rps)],
                        out_hbm.at[c, pl.ds(s * rps, rps)])

    return deg_kernel


GROUP = 16


def _make_conv_kernel(n, n_pad, d, n_chunks_total):
    per_tile = n_chunks_total // NW
    assert per_tile % GROUP == 0
    rps = n_pad // NS
    mesh = plsc.VectorSubcoreMesh(core_axis_name="c", subcore_axis_name="s",
                                  num_cores=NC, num_subcores=NS)

    @functools.partial(
        pl.kernel,
        out_type=jax.ShapeDtypeStruct((NC, n_pad, d), jnp.float32),
        mesh=mesh,
        scratch_types=[
            pltpu.VMEM((GROUP, CHUNK), jnp.int32),
            pltpu.VMEM((GROUP, CHUNK), jnp.int32),
            pltpu.VMEM((2, CHUNK, d), jnp.float32),
            pltpu.VMEM_SHARED((n_pad, d), jnp.float32),
            pltpu.SemaphoreType.DMA,
            pltpu.SemaphoreType.DMA,
        ],
    )
    def conv_kernel(g_hbm, row2d, col2d, zeros_hbm, out_hbm,
                    ridx, cidx, rows, acc, gsem, ssem):
        c = lax.axis_index("c")
        s = lax.axis_index("s")
        tile = c * NS + s
        pltpu.sync_copy(zeros_hbm.at[pl.ds(s * rps, rps)], acc.at[pl.ds(s * rps, rps)])
        plsc.subcore_barrier()

        def outer(gi, carry):
            base = tile * per_tile + gi * GROUP
            pltpu.sync_copy(row2d.at[pl.ds(base, GROUP)], ridx)
            pltpu.sync_copy(col2d.at[pl.ds(base, GROUP)], cidx)
            pltpu.async_copy(g_hbm.at[ridx.at[0]], rows.at[0], gsem)

            def inner(j, c2):
                pltpu.make_async_copy(g_hbm.at[ridx.at[j]], rows.at[j % 2],
                                      gsem).wait()
                pltpu.async_copy(rows.at[j % 2], acc.at[cidx.at[j]], ssem,
                                 add=True)

                @pl.when(j >= 1)
                def _():
                    pltpu.make_async_copy(rows.at[(j + 1) % 2],
                                          acc.at[cidx.at[j]], ssem).wait()

                @pl.when(j + 1 < GROUP)
                def _():
                    pltpu.async_copy(g_hbm.at[ridx.at[j + 1]],
                                     rows.at[(j + 1) % 2], gsem)

                return c2

            lax.fori_loop(0, GROUP, inner, 0)
            pltpu.make_async_copy(rows.at[0], acc.at[cidx.at[0]], ssem).wait()
            return carry

        lax.fori_loop(0, per_tile // GROUP, outer, 0)
        plsc.subcore_barrier()
        pltpu.sync_copy(acc.at[pl.ds(s * rps, rps)],
                        out_hbm.at[c, pl.ds(s * rps, rps)])

    return conv_kernel


def _dis_from_degp(degp_blk):
    deg = degp_blk[0, :, 0:1] + degp_blk[1, :, 0:1] + 1.0
    return lax.rsqrt(deg)


def _prep_body(x_ref, w_ref, degp_ref, g_ref):
    dis = _dis_from_degp(degp_ref)
    g_ref[...] = jnp.dot(x_ref[...], w_ref[...],
                         preferred_element_type=jnp.float32) * dis


def _mid1_body(sp_ref, g_ref, w_ref, b_ref, degp_ref, h_ref, gn_ref):
    dis = _dis_from_degp(degp_ref)
    conv = dis * (sp_ref[0] + sp_ref[1] + g_ref[...]) + b_ref[...]
    h = _elu(conv)
    h_ref[...] = h
    gn_ref[...] = jnp.dot(h, w_ref[...],
                          preferred_element_type=jnp.float32) * dis


def _mid2_body(sp_ref, g_ref, hprev_ref, b_ref, degp_ref, h_ref, u_ref):
    dis = _dis_from_degp(degp_ref)
    conv = dis * (sp_ref[0] + sp_ref[1] + g_ref[...]) + b_ref[...]
    h = _elu(hprev_ref[...] + conv)
    h_ref[...] = h
    u_ref[...] = h * dis


def _make_final_body(n_blocks):
    def _final_body(ap_ref, u_ref, h2r_ref, degp_ref, w3_ref, b3_ref,
                    lng_ref, lnb_ref, w1_ref, b1_ref, w2_ref, b2_ref,
                    out_ref, acc_ref):
        i = pl.program_id(0)

        @pl.when(i == 0)
        def _():
            acc_ref[...] = jnp.zeros_like(acc_ref)

        a = ap_ref[0, :, 0:1] + ap_ref[1, :, 0:1]
        acc_ref[0:1, :] += jnp.sum(u_ref[...] * a, axis=0, keepdims=True)

        @pl.when(i == n_blocks - 1)
        def _():
            deg0 = degp_ref[0, 0, 0] + degp_ref[1, 0, 0] + 1.0
            dis0 = lax.rsqrt(deg0)
            u0 = h2r_ref[0:1, :] * dis0
            t0 = acc_ref[0:1, :] + u0
            conv = dis0 * jnp.dot(t0, w3_ref[...],
                                  preferred_element_type=jnp.float32,
                         precision=lax.Precision.HIGHEST) + b3_ref[...]
            h3 = _elu(h2r_ref[0:1, :] + conv)
            mu = jnp.mean(h3)
            var = jnp.mean((h3 - mu) ** 2)
            hln = (h3 - mu) / jnp.sqrt(var + 1e-5) * lng_ref[...] + lnb_ref[...]
            te = _elu(jnp.dot(hln, w1_ref[...],
                              preferred_element_type=jnp.float32)
                      + b1_ref[...])
            out_ref[...] = jnp.dot(te, w2_ref[...],
                                   preferred_element_type=jnp.float32) \
                + b2_ref[...]

    return _final_body


def kernel(x, edge_index, W1, b1, W2, b2, W3, b3, ln_g, ln_b,
           lin1_W, lin1_b, lin2_W, lin2_b):
    n, d = x.shape
    e = edge_index.shape[1]
    out_dim = lin2_W.shape[1]

    n_pad = ((n + CHUNK + NS * 8 - 1) // (NS * 8)) * (NS * 8)
    grain = NW * 8 * CHUNK
    e_pad = ((e + grain - 1) // grain) * grain
    n_chunks_total = e_pad // CHUNK

    row = edge_index[0]
    col = edge_index[1]
    pad = e_pad - e
    pad_i = jnp.arange(pad, dtype=jnp.int32)
    row_p = jnp.concatenate([row, pad_i % n])
    col_p = jnp.concatenate([col, n + pad_i % (n_pad - n)])
    row2d = row_p.reshape(-1, CHUNK)
    col2d = col_p.reshape(-1, CHUNK)
    all_i = jnp.arange(e_pad, dtype=jnp.int32)
    midx2d = jnp.where(col_p == 0, row_p,
                       n + all_i % (n_pad - n)).reshape(-1, CHUNK)
    zeros_nd = jnp.zeros((n_pad, d), jnp.float32)
    ones_d = jnp.ones((CHUNK, d), jnp.float32)

    deg_sc = _make_deg_kernel(n_pad, d, n_chunks_total)
    conv_sc = _make_conv_kernel(n, n_pad, d, n_chunks_total)

    R = 2000
    grid = (n // R,)
    bs_nd = pl.BlockSpec((R, d), lambda i: (i, 0))
    bs_sp = pl.BlockSpec((NC, R, d), lambda i: (0, i, 0))
    bs_degp = pl.BlockSpec((NC, R, d), lambda i: (0, i, 0))
    bs_w = pl.BlockSpec((d, d), lambda i: (0, 0))
    bs_b = pl.BlockSpec((1, d), lambda i: (0, 0))
    shape_nd = jax.ShapeDtypeStruct((n, d), jnp.float32)

    degp = deg_sc(col2d, ones_d, zeros_nd)

    g1 = pl.pallas_call(
        _prep_body, grid=grid,
        in_specs=[bs_nd, bs_w, bs_degp],
        out_specs=bs_nd, out_shape=shape_nd,
    )(x, W1, degp)

    s1p = conv_sc(g1, row2d, col2d, zeros_nd)

    h1, g2 = pl.pallas_call(
        _mid1_body, grid=grid,
        in_specs=[bs_sp, bs_nd, bs_w, bs_b, bs_degp],
        out_specs=[bs_nd, bs_nd], out_shape=[shape_nd, shape_nd],
    )(s1p, g1, W2, b1.reshape(1, d), degp)

    s2p = conv_sc(g2, row2d, col2d, zeros_nd)

    h2, u = pl.pallas_call(
        _mid2_body, grid=grid,
        in_specs=[bs_sp, bs_nd, bs_nd, bs_b, bs_degp],
        out_specs=[bs_nd, bs_nd], out_shape=[shape_nd, shape_nd],
    )(s2p, g2, h1, b2.reshape(1, d), degp)

    ap = deg_sc(midx2d, ones_d, zeros_nd)

    out = pl.pallas_call(
        _make_final_body(grid[0]), grid=grid,
        in_specs=[
            bs_sp,
            bs_nd,
            pl.BlockSpec((8, d), lambda i: (0, 0)),
            pl.BlockSpec((NC, 8, d), lambda i: (0, 0, 0)),
            bs_w, bs_b, bs_b, bs_b,
            bs_w, bs_b,
            pl.BlockSpec((d, out_dim), lambda i: (0, 0)),
            pl.BlockSpec((1, out_dim), lambda i: (0, 0)),
        ],
        out_specs=pl.BlockSpec((1, out_dim), lambda i: (0, 0)),
        out_shape=jax.ShapeDtypeStruct((1, out_dim), jnp.float32),
        scratch_shapes=[pltpu.VMEM((8, d), jnp.float32)],
    )(ap, u, h2, degp, W3, b3.reshape(1, d), ln_g.reshape(1, d),
      ln_b.reshape(1, d), lin1_W, lin1_b.reshape(1, d),
      lin2_W, lin2_b.reshape(1, out_dim))

    return out


# speedup vs baseline: 1.0018x; 1.0018x over previous
import functools

import jax
import jax.numpy as jnp
from jax import lax
from jax.experimental import pallas as pl
from jax.experimental.pallas import tpu as pltpu
from jax.experimental.pallas import tpu_sc as plsc

NC = 2
NS = 16
NW = NC * NS
CHUNK = 128


def _elu(v):
    return jnp.where(v > 0, v, jnp.exp(jnp.minimum(v, 0.0)) - 1.0)


def _make_deg_kernel(n_pad, d, n_chunks_total):
    per_tile = n_chunks_total // NW
    rps = n_pad // NS
    mesh = plsc.VectorSubcoreMesh(core_axis_name="c", subcore_axis_name="s",
                                  num_cores=NC, num_subcores=NS)

    @functools.partial(
        pl.kernel,
        out_type=jax.ShapeDtypeStruct((NC, n_pad, d), jnp.float32),
        mesh=mesh,
        scratch_types=[
            pltpu.VMEM((per_tile, CHUNK), jnp.int32),
            pltpu.VMEM((CHUNK, d), jnp.float32),
            pltpu.VMEM_SHARED((n_pad, d), jnp.float32),
            pltpu.SemaphoreType.DMA,
        ],
    )
    def deg_kernel(col2d, ones_hbm, zeros_hbm, out_hbm, cidx, ones_v, acc, sem):
        c = lax.axis_index("c")
        s = lax.axis_index("s")
        tile = c * NS + s
        pltpu.sync_copy(zeros_hbm.at[pl.ds(s * rps, rps)], acc.at[pl.ds(s * rps, rps)])
        pltpu.sync_copy(col2d.at[pl.ds(tile * per_tile, per_tile)], cidx)
        pltpu.sync_copy(ones_hbm, ones_v)
        plsc.subcore_barrier()

        def body(j, carry):
            pltpu.async_copy(ones_v, acc.at[cidx.at[j]], sem, add=True)
            return carry

        lax.fori_loop(0, per_tile, body, 0)

        def drain(j, carry):
            pltpu.make_async_copy(ones_v, acc.at[cidx.at[j]], sem).wait()
            return carry

        lax.fori_loop(0, per_tile, drain, 0)
        plsc.subcore_barrier()
        pltpu.sync_copy(acc.at[pl.ds(s * rps, rps)],
                        out_hbm.at[c, pl.ds(s * rps, rps)])

    return deg_kernel


GROUP = 16


def _make_conv_kernel(n, n_pad, d, n_chunks_total):
    per_tile = n_chunks_total // NW
    assert per_tile % GROUP == 0
    rps = n_pad // NS
    mesh = plsc.VectorSubcoreMesh(core_axis_name="c", subcore_axis_name="s",
                                  num_cores=NC, num_subcores=NS)

    @functools.partial(
        pl.kernel,
        out_type=jax.ShapeDtypeStruct((NC, n_pad, d), jnp.float32),
        mesh=mesh,
        scratch_types=[
            pltpu.VMEM((GROUP, CHUNK), jnp.int32),
            pltpu.VMEM((GROUP, CHUNK), jnp.int32),
            pltpu.VMEM((2, CHUNK, d), jnp.float32),
            pltpu.VMEM_SHARED((n_pad, d), jnp.float32),
            pltpu.SemaphoreType.DMA,
            pltpu.SemaphoreType.DMA,
        ],
    )
    def conv_kernel(g_hbm, row2d, col2d, zeros_hbm, out_hbm,
                    ridx, cidx, rows, acc, gsem, ssem):
        c = lax.axis_index("c")
        s = lax.axis_index("s")
        tile = c * NS + s
        pltpu.sync_copy(zeros_hbm.at[pl.ds(s * rps, rps)], acc.at[pl.ds(s * rps, rps)])
        plsc.subcore_barrier()

        def outer(gi, carry):
            base = tile * per_tile + gi * GROUP
            pltpu.sync_copy(row2d.at[pl.ds(base, GROUP)], ridx)
            pltpu.sync_copy(col2d.at[pl.ds(base, GROUP)], cidx)
            pltpu.async_copy(g_hbm.at[ridx.at[0]], rows.at[0], gsem)

            def inner(j, c2):
                pltpu.make_async_copy(g_hbm.at[ridx.at[j]], rows.at[j % 2],
                                      gsem).wait()
                pltpu.async_copy(rows.at[j % 2], acc.at[cidx.at[j]], ssem,
                                 add=True)

                @pl.when(j >= 1)
                def _():
                    pltpu.make_async_copy(rows.at[(j + 1) % 2],
                                          acc.at[cidx.at[j]], ssem).wait()

                @pl.when(j + 1 < GROUP)
                def _():
                    pltpu.async_copy(g_hbm.at[ridx.at[j + 1]],
                                     rows.at[(j + 1) % 2], gsem)

                return c2

            lax.fori_loop(0, GROUP, inner, 0)
            pltpu.make_async_copy(rows.at[0], acc.at[cidx.at[0]], ssem).wait()
            return carry

        lax.fori_loop(0, per_tile // GROUP, outer, 0)
        plsc.subcore_barrier()
        pltpu.sync_copy(acc.at[pl.ds(s * rps, rps)],
                        out_hbm.at[c, pl.ds(s * rps, rps)])

    return conv_kernel


def _dis_from_degp(degp_blk):
    deg = degp_blk[0, :, 0:1] + degp_blk[1, :, 0:1] + 1.0
    return lax.rsqrt(deg)


def _prep_body(x_ref, w_ref, degp_ref, g_ref):
    dis = _dis_from_degp(degp_ref)
    g_ref[...] = jnp.dot(x_ref[...], w_ref[...],
                         preferred_element_type=jnp.float32) * dis


def _mid1_body(sp_ref, g_ref, w_ref, b_ref, degp_ref, h_ref, gn_ref):
    dis = _dis_from_degp(degp_ref)
    conv = dis * (sp_ref[0] + sp_ref[1] + g_ref[...]) + b_ref[...]
    h = _elu(conv)
    h_ref[...] = h
    gn_ref[...] = jnp.dot(h, w_ref[...],
                          preferred_element_type=jnp.float32) * dis


def _mid2_body(sp_ref, g_ref, hprev_ref, w_ref, b_ref, degp_ref, h_ref, u_ref):
    dis = _dis_from_degp(degp_ref)
    conv = dis * (sp_ref[0] + sp_ref[1] + g_ref[...]) + b_ref[...]
    h = _elu(hprev_ref[...] + conv)
    h_ref[...] = h
    u_ref[...] = jnp.dot(h, w_ref[...],
                         preferred_element_type=jnp.float32) * dis


def _make_final_body(n_blocks):
    def _final_body(ap_ref, g3_ref, g30_ref, h2r_ref, degp_ref, b3_ref,
                    lng_ref, lnb_ref, w1_ref, b1_ref, w2_ref, b2_ref,
                    out_ref, acc_ref):
        i = pl.program_id(0)

        @pl.when(i == 0)
        def _():
            acc_ref[...] = jnp.zeros_like(acc_ref)

        a = ap_ref[0, :, 0:1] + ap_ref[1, :, 0:1]
        acc_ref[0:1, :] += jnp.sum(g3_ref[...] * a, axis=0, keepdims=True)

        @pl.when(i == n_blocks - 1)
        def _():
            deg0 = degp_ref[0, 0, 0] + degp_ref[1, 0, 0] + 1.0
            dis0 = lax.rsqrt(deg0)
            t0 = acc_ref[0:1, :] + g30_ref[0:1, :]
            conv = dis0 * t0 + b3_ref[...]
            h3 = _elu(h2r_ref[0:1, :] + conv)
            mu = jnp.mean(h3)
            var = jnp.mean((h3 - mu) ** 2)
            hln = (h3 - mu) / jnp.sqrt(var + 1e-5) * lng_ref[...] + lnb_ref[...]
            te = _elu(jnp.dot(hln, w1_ref[...],
                              preferred_element_type=jnp.float32)
                      + b1_ref[...])
            out_ref[...] = jnp.dot(te, w2_ref[...],
                                   preferred_element_type=jnp.float32) \
                + b2_ref[...]

    return _final_body


def kernel(x, edge_index, W1, b1, W2, b2, W3, b3, ln_g, ln_b,
           lin1_W, lin1_b, lin2_W, lin2_b):
    n, d = x.shape
    e = edge_index.shape[1]
    out_dim = lin2_W.shape[1]

    n_pad = ((n + CHUNK + NS * 8 - 1) // (NS * 8)) * (NS * 8)
    grain = NW * 8 * CHUNK
    e_pad = ((e + grain - 1) // grain) * grain
    n_chunks_total = e_pad // CHUNK

    row = edge_index[0]
    col = edge_index[1]
    pad = e_pad - e
    pad_i = jnp.arange(pad, dtype=jnp.int32)
    row_p = jnp.concatenate([row, pad_i % n])
    col_p = jnp.concatenate([col, n + pad_i % (n_pad - n)])
    row2d = row_p.reshape(-1, CHUNK)
    col2d = col_p.reshape(-1, CHUNK)
    all_i = jnp.arange(e_pad, dtype=jnp.int32)
    midx2d = jnp.where(col_p == 0, row_p,
                       n + all_i % (n_pad - n)).reshape(-1, CHUNK)
    zeros_nd = jnp.zeros((n_pad, d), jnp.float32)
    ones_d = jnp.ones((CHUNK, d), jnp.float32)

    deg_sc = _make_deg_kernel(n_pad, d, n_chunks_total)
    conv_sc = _make_conv_kernel(n, n_pad, d, n_chunks_total)

    R = 2000
    grid = (n // R,)
    bs_nd = pl.BlockSpec((R, d), lambda i: (i, 0))
    bs_sp = pl.BlockSpec((NC, R, d), lambda i: (0, i, 0))
    bs_degp = pl.BlockSpec((NC, R, d), lambda i: (0, i, 0))
    bs_w = pl.BlockSpec((d, d), lambda i: (0, 0))
    bs_b = pl.BlockSpec((1, d), lambda i: (0, 0))
    shape_nd = jax.ShapeDtypeStruct((n, d), jnp.float32)

    degp = deg_sc(col2d, ones_d, zeros_nd)

    g1 = pl.pallas_call(
        _prep_body, grid=grid,
        in_specs=[bs_nd, bs_w, bs_degp],
        out_specs=bs_nd, out_shape=shape_nd,
    )(x, W1, degp)

    s1p = conv_sc(g1, row2d, col2d, zeros_nd)

    h1, g2 = pl.pallas_call(
        _mid1_body, grid=grid,
        in_specs=[bs_sp, bs_nd, bs_w, bs_b, bs_degp],
        out_specs=[bs_nd, bs_nd], out_shape=[shape_nd, shape_nd],
    )(s1p, g1, W2, b1.reshape(1, d), degp)

    s2p = conv_sc(g2, row2d, col2d, zeros_nd)

    h2, g3 = pl.pallas_call(
        _mid2_body, grid=grid,
        in_specs=[bs_sp, bs_nd, bs_nd, bs_w, bs_b, bs_degp],
        out_specs=[bs_nd, bs_nd], out_shape=[shape_nd, shape_nd],
    )(s2p, g2, h1, W3, b2.reshape(1, d), degp)

    ap = deg_sc(midx2d, ones_d, zeros_nd)

    out = pl.pallas_call(
        _make_final_body(grid[0]), grid=grid,
        in_specs=[
            bs_sp,
            bs_nd,
            pl.BlockSpec((8, d), lambda i: (0, 0)),
            pl.BlockSpec((8, d), lambda i: (0, 0)),
            pl.BlockSpec((NC, 8, d), lambda i: (0, 0, 0)),
            bs_b, bs_b, bs_b,
            bs_w, bs_b,
            pl.BlockSpec((d, out_dim), lambda i: (0, 0)),
            pl.BlockSpec((1, out_dim), lambda i: (0, 0)),
        ],
        out_specs=pl.BlockSpec((1, out_dim), lambda i: (0, 0)),
        out_shape=jax.ShapeDtypeStruct((1, out_dim), jnp.float32),
        scratch_shapes=[pltpu.VMEM((8, d), jnp.float32)],
    )(ap, g3, g3, h2, degp, b3.reshape(1, d), ln_g.reshape(1, d),
      ln_b.reshape(1, d), lin1_W, lin1_b.reshape(1, d),
      lin2_W, lin2_b.reshape(1, out_dim))

    return out
